# baseline (speedup 1.0000x reference)
import functools

import jax
import jax.numpy as jnp
from jax import lax
from jax.experimental import pallas as pl
from jax.experimental.pallas import tpu as pltpu
from jax.experimental.pallas import tpu_sc as plsc

NSEG = 256


def _argmin_body(nk, bk, x_ref, cc_ref, b_ref, idx_ref, sumx_ref, cnt_ref,
                 minval, minidx, mm2, c2s, x2s):
    i = pl.program_id(0)
    j = pl.program_id(1)
    x = x_ref[...]
    bn = x.shape[0]

    @pl.when(jnp.logical_and(i == 0, j == 0))
    def _():
        sumx_ref[...] = jnp.zeros_like(sumx_ref)
        cnt_ref[...] = jnp.zeros_like(cnt_ref)

    @pl.when(j == 0)
    def _():
        b2 = b_ref[0]
        seg = lax.broadcasted_iota(jnp.int32, (bn, NSEG), 1)
        mask_t = (b2 == seg).astype(jnp.float32)
        sumx_ref[...] += lax.dot_general(
            mask_t, x, (((0,), (0,)), ((), ())),
            preferred_element_type=jnp.float32)
        cnt_ref[...] += lax.dot_general(
            mask_t, jnp.ones((bn, 1), jnp.float32), (((0,), (0,)), ((), ())),
            preferred_element_type=jnp.float32)
        x2s[...] = jnp.sum(x * x, axis=1, keepdims=True)

    @pl.when(j < nk)
    def _():
        cc = cc_ref[...]

        @pl.when(i == 0)
        def _():
            c2s[j] = jnp.sum(cc * cc, axis=1)

        mm2[j % 2] = lax.dot_general(x, cc, (((1,), (1,)), ((), ())),
                                     preferred_element_type=jnp.float32)

    @pl.when(j > 0)
    def _():
        jj = j - 1
        mm = mm2[(j + 1) % 2]
        c2 = c2s[jj]
        scores = x2s[...] + c2[None, :] - 2.0 * mm
        bm = jnp.min(scores, axis=1, keepdims=True)
        col = lax.broadcasted_iota(jnp.int32, scores.shape, 1)
        bidx = jnp.min(jnp.where(scores == bm, col, jnp.int32(2 ** 30)),
                       axis=1, keepdims=True) + jj * bk

        @pl.when(jj == 0)
        def _():
            minval[...] = bm
            minidx[...] = bidx

        @pl.when(jj > 0)
        def _():
            better = bm < minval[...]
            minval[...] = jnp.where(better, bm, minval[...])
            minidx[...] = jnp.where(better, bidx, minidx[...])

        @pl.when(jj == nk - 1)
        def _():
            idx_ref[0] = minidx[...]


def _argmin_call(x, cc, batch, bn, bk):
    n, d = x.shape
    k = cc.shape[0]
    nn, nk = n // bn, k // bk
    b3 = batch.reshape(nn, bn, 1)
    return pl.pallas_call(
        functools.partial(_argmin_body, nk, bk),
        grid=(nn, nk + 1),
        in_specs=[
            pl.BlockSpec((bn, d), lambda i, j: (i, 0)),
            pl.BlockSpec((bk, d), lambda i, j: (jnp.minimum(j, nk - 1), 0)),
            pl.BlockSpec((1, bn, 1), lambda i, j: (i, 0, 0)),
        ],
        out_specs=[
            pl.BlockSpec((1, bn, 1), lambda i, j: (i, 0, 0)),
            pl.BlockSpec((NSEG, d), lambda i, j: (0, 0)),
            pl.BlockSpec((NSEG, 1), lambda i, j: (0, 0)),
        ],
        out_shape=[
            jax.ShapeDtypeStruct((nn, bn, 1), jnp.int32),
            jax.ShapeDtypeStruct((NSEG, d), jnp.float32),
            jax.ShapeDtypeStruct((NSEG, 1), jnp.float32),
        ],
        scratch_shapes=[
            pltpu.VMEM((bn, 1), jnp.float32),
            pltpu.VMEM((bn, 1), jnp.int32),
            pltpu.VMEM((2, bn, bk), jnp.float32),
            pltpu.VMEM((nk, bk), jnp.float32),
            pltpu.VMEM((bn, 1), jnp.float32),
        ],
    )(x, cc, b3)


def _sc_gather3(idx, t0, t1, t2):
    n = idx.shape[0]
    d = t0.shape[1]
    nw = 32
    rw = 320
    ch = 80
    assert n == 31 * rw + ch and rw % ch == 0

    mesh = plsc.VectorSubcoreMesh(core_axis_name="c", subcore_axis_name="s",
                                  num_cores=2, num_subcores=16)
    out_t = [jax.ShapeDtypeStruct((n, d), jnp.float32)] * 3

    @functools.partial(
        pl.kernel, out_type=out_t, mesh=mesh,
        scratch_types=[
            pltpu.VMEM((ch,), jnp.int32),
            pltpu.VMEM((ch, d), jnp.float32),
            pltpu.VMEM((ch, d), jnp.float32),
            pltpu.VMEM((ch, d), jnp.float32),
            pltpu.SemaphoreType.DMA,
            pltpu.SemaphoreType.DMA,
            pltpu.SemaphoreType.DMA,
        ],
    )
    def k(idx_hbm, t0_hbm, t1_hbm, t2_hbm, o0_hbm, o1_hbm, o2_hbm,
          idx_v, r0, r1, r2, s0, s1, s2):
        wid = lax.axis_index("s") * 2 + lax.axis_index("c")
        base = wid * rw
        for c in range(rw // ch):
            off = jnp.minimum(base + c * ch, n - ch)
            pltpu.sync_copy(idx_hbm.at[pl.ds(off, ch)], idx_v)
            cp0 = pltpu.make_async_copy(t0_hbm.at[idx_v], r0, s0)
            cp1 = pltpu.make_async_copy(t1_hbm.at[idx_v], r1, s1)
            cp2 = pltpu.make_async_copy(t2_hbm.at[idx_v], r2, s2)
            cp0.start(); cp1.start(); cp2.start()
            cp0.wait(); cp1.wait(); cp2.wait()
            pltpu.sync_copy(r0, o0_hbm.at[pl.ds(off, ch)])
            pltpu.sync_copy(r1, o1_hbm.at[pl.ds(off, ch)])
            pltpu.sync_copy(r2, o2_hbm.at[pl.ds(off, ch)])

    return k(idx, t0, t1, t2)


def _pool_body(nn, causal_ref, counter_ref, b_ref, sumx_ref, cnt_ref,
               w_ref, bias_ref,
               cpre_ref, kpre_ref, ypre_ref, pc_ref, px_ref,
               acc_c, acc_k):
    i = pl.program_id(0)
    b2 = b_ref[0]
    bn = b2.shape[0]
    seg = lax.broadcasted_iota(jnp.int32, (bn, NSEG), 1)
    mask_t = (b2 == seg).astype(jnp.float32)

    @pl.when(i == 0)
    def _():
        acc_c[...] = jnp.zeros_like(acc_c)
        acc_k[...] = jnp.zeros_like(acc_k)

    acc_c[...] += lax.dot_general(mask_t, causal_ref[...],
                                  (((0,), (0,)), ((), ())),
                                  preferred_element_type=jnp.float32)
    acc_k[...] += lax.dot_general(mask_t, counter_ref[...],
                                  (((0,), (0,)), ((), ())),
                                  preferred_element_type=jnp.float32)

    @pl.when(i == nn - 1)
    def _():
        cnt = jnp.maximum(cnt_ref[...], 1.0)
        pooled_x = sumx_ref[...] / cnt
        pooled_c = pooled_x + acc_c[...] / cnt
        pooled_k = acc_k[...] / cnt
        w = w_ref[...]
        bias = bias_ref[...]
        dn = (((1,), (1,)), ((), ()))
        cpre_ref[...] = lax.dot_general(
            pooled_c, w, dn, preferred_element_type=jnp.float32) + bias
        kpre_ref[...] = lax.dot_general(
            pooled_k, w, dn, preferred_element_type=jnp.float32) + bias
        ypre_ref[...] = lax.dot_general(
            pooled_x, w, dn, preferred_element_type=jnp.float32) + bias
        pc_ref[...] = pooled_c
        px_ref[...] = pooled_x


def _pool_call(causal_rows, counter_rows, batch, sumx, cnt, w, bias, bn):
    n, d = causal_rows.shape
    t = w.shape[0]
    nn = n // bn
    b3 = batch.reshape(nn, bn, 1)
    whole = lambda shape: pl.BlockSpec(shape, lambda i: tuple(0 for _ in shape))
    return pl.pallas_call(
        functools.partial(_pool_body, nn),
        grid=(nn,),
        in_specs=[
            pl.BlockSpec((bn, d), lambda i: (i, 0)),
            pl.BlockSpec((bn, d), lambda i: (i, 0)),
            pl.BlockSpec((1, bn, 1), lambda i: (i, 0, 0)),
            whole((NSEG, d)),
            whole((NSEG, 1)),
            whole((t, d)),
            whole((1, t)),
        ],
        out_specs=[
            whole((NSEG, t)),
            whole((NSEG, t)),
            whole((NSEG, t)),
            whole((NSEG, d)),
            whole((NSEG, d)),
        ],
        out_shape=[
            jax.ShapeDtypeStruct((NSEG, t), jnp.float32),
            jax.ShapeDtypeStruct((NSEG, t), jnp.float32),
            jax.ShapeDtypeStruct((NSEG, t), jnp.float32),
            jax.ShapeDtypeStruct((NSEG, d), jnp.float32),
            jax.ShapeDtypeStruct((NSEG, d), jnp.float32),
        ],
        scratch_shapes=[
            pltpu.VMEM((NSEG, d), jnp.float32),
            pltpu.VMEM((NSEG, d), jnp.float32),
        ],
    )(causal_rows, counter_rows, b3, sumx, cnt, w, bias)


def kernel(x, batch, codebook, causal_codebook, counter_codebook, W, b):
    n, d = x.shape
    batch = batch.astype(jnp.int32)
    bn = 1000
    bk = 1024

    idx3, sumx, cnt = _argmin_call(x, causal_codebook, batch, bn, bk)
    idx = idx3.reshape(n)

    z_nodes, causal_rows, counter_rows = _sc_gather3(
        idx, codebook, causal_codebook, counter_codebook)

    causal_pre, counter_pre, y_pre, pooled_causal, pooled_x = _pool_call(
        causal_rows, counter_rows, batch, sumx, cnt, W,
        b.reshape(1, -1), bn)

    return (causal_pre, counter_pre, y_pre, z_nodes, pooled_causal, pooled_x)


# speedup vs baseline: 1.1031x; 1.1031x over previous
import functools

import jax
import jax.numpy as jnp
from jax import lax
from jax.experimental import pallas as pl
from jax.experimental.pallas import tpu as pltpu
from jax.experimental.pallas import tpu_sc as plsc

NSEG = 256


def _argmin_body(nk, bk, x_ref, cc_ref, b_ref, idx_ref, sumx_ref, cnt_ref,
                 minval, minidx, mm_ref, c2s, x2s):
    i = pl.program_id(0)
    j = pl.program_id(1)
    x = x_ref[...]
    bn = x.shape[0]

    @pl.when(jnp.logical_and(i == 0, j == 0))
    def _():
        sumx_ref[...] = jnp.zeros_like(sumx_ref)
        cnt_ref[...] = jnp.zeros_like(cnt_ref)

    @pl.when(j == 0)
    def _():
        b2 = b_ref[0]
        seg = lax.broadcasted_iota(jnp.int32, (bn, NSEG), 1)
        mask_t = (b2 == seg).astype(jnp.float32)
        sumx_ref[...] += lax.dot_general(
            mask_t, x, (((0,), (0,)), ((), ())),
            preferred_element_type=jnp.float32)
        cnt_ref[...] += lax.dot_general(
            mask_t, jnp.ones((bn, 1), jnp.float32), (((0,), (0,)), ((), ())),
            preferred_element_type=jnp.float32)
        x2s[...] = jnp.sum(x * x, axis=1, keepdims=True)

    @pl.when(jnp.logical_and(i == 0, j < nk))
    def _():
        c2s[j] = jnp.sum(cc_ref[...] * cc_ref[...], axis=1)

    jj = j - 1
    mm = mm_ref[...]
    c2 = c2s[jnp.maximum(jj, 0)]
    scores = x2s[...] + c2[None, :] - 2.0 * mm
    bm = jnp.min(scores, axis=1, keepdims=True)
    col = lax.broadcasted_iota(jnp.int32, scores.shape, 1)
    bidx = jnp.min(jnp.where(scores == bm, col, jnp.int32(2 ** 30)),
                   axis=1, keepdims=True) + jj * bk
    first = (jj == 0)
    better = jnp.logical_or(first, bm < minval[...])
    minval[...] = jnp.where(better, bm, minval[...])
    minidx[...] = jnp.where(better, bidx, minidx[...])
    mm_ref[...] = lax.dot_general(x, cc_ref[...], (((1,), (1,)), ((), ())),
                                  preferred_element_type=jnp.float32)

    @pl.when(jj == nk - 1)
    def _():
        idx_ref[0] = minidx[...]


def _argmin_call(x, cc, batch, bn, bk):
    n, d = x.shape
    k = cc.shape[0]
    nn, nk = n // bn, k // bk
    b3 = batch.reshape(nn, bn, 1)
    return pl.pallas_call(
        functools.partial(_argmin_body, nk, bk),
        grid=(nn, nk + 1),
        in_specs=[
            pl.BlockSpec((bn, d), lambda i, j: (i, 0)),
            pl.BlockSpec((bk, d), lambda i, j: (jnp.minimum(j, nk - 1), 0)),
            pl.BlockSpec((1, bn, 1), lambda i, j: (i, 0, 0)),
        ],
        out_specs=[
            pl.BlockSpec((1, bn, 1), lambda i, j: (i, 0, 0)),
            pl.BlockSpec((NSEG, d), lambda i, j: (0, 0)),
            pl.BlockSpec((NSEG, 1), lambda i, j: (0, 0)),
        ],
        out_shape=[
            jax.ShapeDtypeStruct((nn, bn, 1), jnp.int32),
            jax.ShapeDtypeStruct((NSEG, d), jnp.float32),
            jax.ShapeDtypeStruct((NSEG, 1), jnp.float32),
        ],
        scratch_shapes=[
            pltpu.VMEM((bn, 1), jnp.float32),
            pltpu.VMEM((bn, 1), jnp.int32),
            pltpu.VMEM((bn, bk), jnp.float32),
            pltpu.VMEM((nk, bk), jnp.float32),
            pltpu.VMEM((bn, 1), jnp.float32),
        ],
    )(x, cc, b3)


def _sc_gather3(idx, t0, t1, t2):
    n = idx.shape[0]
    d = t0.shape[1]
    nw = 32
    rw = 320
    ch = 80
    assert n == 31 * rw + ch and rw % ch == 0

    mesh = plsc.VectorSubcoreMesh(core_axis_name="c", subcore_axis_name="s",
                                  num_cores=2, num_subcores=16)
    out_t = [jax.ShapeDtypeStruct((n, d), jnp.float32)] * 3

    @functools.partial(
        pl.kernel, out_type=out_t, mesh=mesh,
        scratch_types=[
            pltpu.VMEM((ch,), jnp.int32),
            pltpu.VMEM((ch, d), jnp.float32),
            pltpu.VMEM((ch, d), jnp.float32),
            pltpu.VMEM((ch, d), jnp.float32),
            pltpu.SemaphoreType.DMA,
            pltpu.SemaphoreType.DMA,
            pltpu.SemaphoreType.DMA,
        ],
    )
    def k(idx_hbm, t0_hbm, t1_hbm, t2_hbm, o0_hbm, o1_hbm, o2_hbm,
          idx_v, r0, r1, r2, s0, s1, s2):
        wid = lax.axis_index("s") * 2 + lax.axis_index("c")
        base = wid * rw
        for c in range(rw // ch):
            off = jnp.minimum(base + c * ch, n - ch)
            pltpu.sync_copy(idx_hbm.at[pl.ds(off, ch)], idx_v)
            cp0 = pltpu.make_async_copy(t0_hbm.at[idx_v], r0, s0)
            cp1 = pltpu.make_async_copy(t1_hbm.at[idx_v], r1, s1)
            cp2 = pltpu.make_async_copy(t2_hbm.at[idx_v], r2, s2)
            cp0.start(); cp1.start(); cp2.start()
            cp0.wait(); cp1.wait(); cp2.wait()
            pltpu.sync_copy(r0, o0_hbm.at[pl.ds(off, ch)])
            pltpu.sync_copy(r1, o1_hbm.at[pl.ds(off, ch)])
            pltpu.sync_copy(r2, o2_hbm.at[pl.ds(off, ch)])

    return k(idx, t0, t1, t2)


def _pool_body(nn, causal_ref, counter_ref, b_ref, sumx_ref, cnt_ref,
               w_ref, bias_ref,
               cpre_ref, kpre_ref, ypre_ref, pc_ref, px_ref,
               acc_c, acc_k):
    i = pl.program_id(0)
    b2 = b_ref[0]
    bn = b2.shape[0]
    seg = lax.broadcasted_iota(jnp.int32, (bn, NSEG), 1)
    mask_t = (b2 == seg).astype(jnp.float32)

    @pl.when(i == 0)
    def _():
        acc_c[...] = jnp.zeros_like(acc_c)
        acc_k[...] = jnp.zeros_like(acc_k)

    acc_c[...] += lax.dot_general(mask_t, causal_ref[...],
                                  (((0,), (0,)), ((), ())),
                                  preferred_element_type=jnp.float32)
    acc_k[...] += lax.dot_general(mask_t, counter_ref[...],
                                  (((0,), (0,)), ((), ())),
                                  preferred_element_type=jnp.float32)

    @pl.when(i == nn - 1)
    def _():
        cnt = jnp.maximum(cnt_ref[...], 1.0)
        pooled_x = sumx_ref[...] / cnt
        pooled_c = pooled_x + acc_c[...] / cnt
        pooled_k = acc_k[...] / cnt
        w = w_ref[...]
        bias = bias_ref[...]
        dn = (((1,), (1,)), ((), ()))
        cpre_ref[...] = lax.dot_general(
            pooled_c, w, dn, preferred_element_type=jnp.float32) + bias
        kpre_ref[...] = lax.dot_general(
            pooled_k, w, dn, preferred_element_type=jnp.float32) + bias
        ypre_ref[...] = lax.dot_general(
            pooled_x, w, dn, preferred_element_type=jnp.float32) + bias
        pc_ref[...] = pooled_c
        px_ref[...] = pooled_x


def _pool_call(causal_rows, counter_rows, batch, sumx, cnt, w, bias, bn):
    n, d = causal_rows.shape
    t = w.shape[0]
    nn = n // bn
    b3 = batch.reshape(nn, bn, 1)
    whole = lambda shape: pl.BlockSpec(shape, lambda i: tuple(0 for _ in shape))
    return pl.pallas_call(
        functools.partial(_pool_body, nn),
        grid=(nn,),
        in_specs=[
            pl.BlockSpec((bn, d), lambda i: (i, 0)),
            pl.BlockSpec((bn, d), lambda i: (i, 0)),
            pl.BlockSpec((1, bn, 1), lambda i: (i, 0, 0)),
            whole((NSEG, d)),
            whole((NSEG, 1)),
            whole((t, d)),
            whole((1, t)),
        ],
        out_specs=[
            whole((NSEG, t)),
            whole((NSEG, t)),
            whole((NSEG, t)),
            whole((NSEG, d)),
            whole((NSEG, d)),
        ],
        out_shape=[
            jax.ShapeDtypeStruct((NSEG, t), jnp.float32),
            jax.ShapeDtypeStruct((NSEG, t), jnp.float32),
            jax.ShapeDtypeStruct((NSEG, t), jnp.float32),
            jax.ShapeDtypeStruct((NSEG, d), jnp.float32),
            jax.ShapeDtypeStruct((NSEG, d), jnp.float32),
        ],
        scratch_shapes=[
            pltpu.VMEM((NSEG, d), jnp.float32),
            pltpu.VMEM((NSEG, d), jnp.float32),
        ],
    )(causal_rows, counter_rows, b3, sumx, cnt, w, bias)


def kernel(x, batch, codebook, causal_codebook, counter_codebook, W, b):
    n, d = x.shape
    batch = batch.astype(jnp.int32)
    bn = 1000
    bk = 1024

    idx3, sumx, cnt = _argmin_call(x, causal_codebook, batch, bn, bk)
    idx = idx3.reshape(n)

    z_nodes, causal_rows, counter_rows = _sc_gather3(
        idx, codebook, causal_codebook, counter_codebook)

    causal_pre, counter_pre, y_pre, pooled_causal, pooled_x = _pool_call(
        causal_rows, counter_rows, batch, sumx, cnt, W,
        b.reshape(1, -1), bn)

    return (causal_pre, counter_pre, y_pre, z_nodes, pooled_causal, pooled_x)


# speedup vs baseline: 1.1638x; 1.0550x over previous
import functools

import jax
import jax.numpy as jnp
from jax import lax
from jax.experimental import pallas as pl
from jax.experimental.pallas import tpu as pltpu
from jax.experimental.pallas import tpu_sc as plsc

NSEG = 256


def _xstats_body(nn, x_ref, b_ref, x2_ref, sumx_ref, cnt_ref):
    i = pl.program_id(0)
    x = x_ref[...]
    bn = x.shape[0]
    x2_ref[...] = jnp.sum(x * x, axis=1, keepdims=True)

    @pl.when(i == 0)
    def _():
        sumx_ref[...] = jnp.zeros_like(sumx_ref)
        cnt_ref[...] = jnp.zeros_like(cnt_ref)

    b2 = b_ref[0]
    seg = lax.broadcasted_iota(jnp.int32, (bn, NSEG), 1)
    mask_t = (b2 == seg).astype(jnp.float32)
    sumx_ref[...] += lax.dot_general(
        mask_t, x, (((0,), (0,)), ((), ())),
        preferred_element_type=jnp.float32)
    cnt_ref[...] += lax.dot_general(
        mask_t, jnp.ones((bn, 1), jnp.float32), (((0,), (0,)), ((), ())),
        preferred_element_type=jnp.float32)


def _xstats_call(x, batch, bn):
    n, d = x.shape
    nn = n // bn
    b3 = batch.reshape(nn, bn, 1)
    return pl.pallas_call(
        functools.partial(_xstats_body, nn),
        grid=(nn,),
        in_specs=[
            pl.BlockSpec((bn, d), lambda i: (i, 0)),
            pl.BlockSpec((1, bn, 1), lambda i: (i, 0, 0)),
        ],
        out_specs=[
            pl.BlockSpec((bn, 1), lambda i: (i, 0)),
            pl.BlockSpec((NSEG, d), lambda i: (0, 0)),
            pl.BlockSpec((NSEG, 1), lambda i: (0, 0)),
        ],
        out_shape=[
            jax.ShapeDtypeStruct((n, 1), jnp.float32),
            jax.ShapeDtypeStruct((NSEG, d), jnp.float32),
            jax.ShapeDtypeStruct((NSEG, 1), jnp.float32),
        ],
    )(x, b3)


def _c2_body(cc_ref, c2_ref):
    cc = cc_ref[...]
    c2_ref[...] = jnp.sum(cc * cc, axis=1)[None, :]


def _c2_call(cc, bk):
    k, d = cc.shape
    nk = k // bk
    return pl.pallas_call(
        _c2_body,
        grid=(nk,),
        in_specs=[pl.BlockSpec((bk, d), lambda j: (j, 0))],
        out_specs=pl.BlockSpec((1, bk), lambda j: (0, j)),
        out_shape=jax.ShapeDtypeStruct((1, k), jnp.float32),
    )(cc)


def _argmin_body(nk, bk, x_ref, cc_ref, x2_ref, c2_ref, idx_ref,
                 minval, minidx):
    j = pl.program_id(1)
    mm = lax.dot_general(x_ref[...], cc_ref[...], (((1,), (1,)), ((), ())),
                         preferred_element_type=jnp.float32)
    scores = x2_ref[...] + c2_ref[...] - 2.0 * mm
    bm = jnp.min(scores, axis=1, keepdims=True)
    col = lax.broadcasted_iota(jnp.int32, scores.shape, 1)
    bidx = jnp.min(jnp.where(scores == bm, col, jnp.int32(2 ** 30)),
                   axis=1, keepdims=True) + j * bk
    better = jnp.logical_or(j == 0, bm < minval[...])
    nv = jnp.where(better, bm, minval[...])
    ni = jnp.where(better, bidx, minidx[...])
    minval[...] = nv
    minidx[...] = ni
    idx_ref[0] = ni


def _argmin_call(x, cc, x2, c2, bn, bk):
    n, d = x.shape
    k = cc.shape[0]
    nn, nk = n // bn, k // bk
    return pl.pallas_call(
        functools.partial(_argmin_body, nk, bk),
        grid=(nn, nk),
        in_specs=[
            pl.BlockSpec((bn, d), lambda i, j: (i, 0)),
            pl.BlockSpec((bk, d), lambda i, j: (j, 0)),
            pl.BlockSpec((bn, 1), lambda i, j: (i, 0)),
            pl.BlockSpec((1, bk), lambda i, j: (0, j)),
        ],
        out_specs=pl.BlockSpec((1, bn, 1), lambda i, j: (i, 0, 0)),
        out_shape=jax.ShapeDtypeStruct((nn, bn, 1), jnp.int32),
        scratch_shapes=[
            pltpu.VMEM((bn, 1), jnp.float32),
            pltpu.VMEM((bn, 1), jnp.int32),
        ],
    )(x, cc, x2, c2)


def _sc_gather3(idx, t0, t1, t2):
    n = idx.shape[0]
    d = t0.shape[1]
    nw = 32
    rw = 320
    ch = 80
    assert n == 31 * rw + ch and rw % ch == 0

    mesh = plsc.VectorSubcoreMesh(core_axis_name="c", subcore_axis_name="s",
                                  num_cores=2, num_subcores=16)
    out_t = [jax.ShapeDtypeStruct((n, d), jnp.float32)] * 3

    @functools.partial(
        pl.kernel, out_type=out_t, mesh=mesh,
        scratch_types=[
            pltpu.VMEM((ch,), jnp.int32),
            pltpu.VMEM((ch, d), jnp.float32),
            pltpu.VMEM((ch, d), jnp.float32),
            pltpu.VMEM((ch, d), jnp.float32),
            pltpu.SemaphoreType.DMA,
            pltpu.SemaphoreType.DMA,
            pltpu.SemaphoreType.DMA,
        ],
    )
    def k(idx_hbm, t0_hbm, t1_hbm, t2_hbm, o0_hbm, o1_hbm, o2_hbm,
          idx_v, r0, r1, r2, s0, s1, s2):
        wid = lax.axis_index("s") * 2 + lax.axis_index("c")
        base = wid * rw
        for c in range(rw // ch):
            off = jnp.minimum(base + c * ch, n - ch)
            pltpu.sync_copy(idx_hbm.at[pl.ds(off, ch)], idx_v)
            cp0 = pltpu.make_async_copy(t0_hbm.at[idx_v], r0, s0)
            cp1 = pltpu.make_async_copy(t1_hbm.at[idx_v], r1, s1)
            cp2 = pltpu.make_async_copy(t2_hbm.at[idx_v], r2, s2)
            cp0.start(); cp1.start(); cp2.start()
            cp0.wait(); cp1.wait(); cp2.wait()
            pltpu.sync_copy(r0, o0_hbm.at[pl.ds(off, ch)])
            pltpu.sync_copy(r1, o1_hbm.at[pl.ds(off, ch)])
            pltpu.sync_copy(r2, o2_hbm.at[pl.ds(off, ch)])

    return k(idx, t0, t1, t2)


def _pool_body(nn, causal_ref, counter_ref, b_ref, sumx_ref, cnt_ref,
               w_ref, bias_ref,
               cpre_ref, kpre_ref, ypre_ref, pc_ref, px_ref,
               acc_c, acc_k):
    i = pl.program_id(0)
    b2 = b_ref[0]
    bn = b2.shape[0]
    seg = lax.broadcasted_iota(jnp.int32, (bn, NSEG), 1)
    mask_t = (b2 == seg).astype(jnp.float32)

    @pl.when(i == 0)
    def _():
        acc_c[...] = jnp.zeros_like(acc_c)
        acc_k[...] = jnp.zeros_like(acc_k)

    acc_c[...] += lax.dot_general(mask_t, causal_ref[...],
                                  (((0,), (0,)), ((), ())),
                                  preferred_element_type=jnp.float32)
    acc_k[...] += lax.dot_general(mask_t, counter_ref[...],
                                  (((0,), (0,)), ((), ())),
                                  preferred_element_type=jnp.float32)

    @pl.when(i == nn - 1)
    def _():
        cnt = jnp.maximum(cnt_ref[...], 1.0)
        pooled_x = sumx_ref[...] / cnt
        pooled_c = pooled_x + acc_c[...] / cnt
        pooled_k = acc_k[...] / cnt
        w = w_ref[...]
        bias = bias_ref[...]
        dn = (((1,), (1,)), ((), ()))
        cpre_ref[...] = lax.dot_general(
            pooled_c, w, dn, preferred_element_type=jnp.float32) + bias
        kpre_ref[...] = lax.dot_general(
            pooled_k, w, dn, preferred_element_type=jnp.float32) + bias
        ypre_ref[...] = lax.dot_general(
            pooled_x, w, dn, preferred_element_type=jnp.float32) + bias
        pc_ref[...] = pooled_c
        px_ref[...] = pooled_x


def _pool_call(causal_rows, counter_rows, batch, sumx, cnt, w, bias, bn):
    n, d = causal_rows.shape
    t = w.shape[0]
    nn = n // bn
    b3 = batch.reshape(nn, bn, 1)
    whole = lambda shape: pl.BlockSpec(shape, lambda i: tuple(0 for _ in shape))
    return pl.pallas_call(
        functools.partial(_pool_body, nn),
        grid=(nn,),
        in_specs=[
            pl.BlockSpec((bn, d), lambda i: (i, 0)),
            pl.BlockSpec((bn, d), lambda i: (i, 0)),
            pl.BlockSpec((1, bn, 1), lambda i: (i, 0, 0)),
            whole((NSEG, d)),
            whole((NSEG, 1)),
            whole((t, d)),
            whole((1, t)),
        ],
        out_specs=[
            whole((NSEG, t)),
            whole((NSEG, t)),
            whole((NSEG, t)),
            whole((NSEG, d)),
            whole((NSEG, d)),
        ],
        out_shape=[
            jax.ShapeDtypeStruct((NSEG, t), jnp.float32),
            jax.ShapeDtypeStruct((NSEG, t), jnp.float32),
            jax.ShapeDtypeStruct((NSEG, t), jnp.float32),
            jax.ShapeDtypeStruct((NSEG, d), jnp.float32),
            jax.ShapeDtypeStruct((NSEG, d), jnp.float32),
        ],
        scratch_shapes=[
            pltpu.VMEM((NSEG, d), jnp.float32),
            pltpu.VMEM((NSEG, d), jnp.float32),
        ],
    )(causal_rows, counter_rows, b3, sumx, cnt, w, bias)


def kernel(x, batch, codebook, causal_codebook, counter_codebook, W, b):
    n, d = x.shape
    batch = batch.astype(jnp.int32)
    bn = 1000
    bk = 1024

    x2, sumx, cnt = _xstats_call(x, batch, bn)
    c2 = _c2_call(causal_codebook, bk)
    idx3 = _argmin_call(x, causal_codebook, x2, c2, bn, bk)
    idx = idx3.reshape(n)

    z_nodes, causal_rows, counter_rows = _sc_gather3(
        idx, codebook, causal_codebook, counter_codebook)

    causal_pre, counter_pre, y_pre, pooled_causal, pooled_x = _pool_call(
        causal_rows, counter_rows, batch, sumx, cnt, W,
        b.reshape(1, -1), bn)

    return (causal_pre, counter_pre, y_pre, z_nodes, pooled_causal, pooled_x)


# speedup vs baseline: 1.1901x; 1.0226x over previous
import functools

import jax
import jax.numpy as jnp
from jax import lax
from jax.experimental import pallas as pl
from jax.experimental.pallas import tpu as pltpu
from jax.experimental.pallas import tpu_sc as plsc

NSEG = 256


def _xstats_body(nn, x_ref, b_ref, x2_ref, sumx_ref, cnt_ref):
    i = pl.program_id(0)
    x = x_ref[...]
    bn = x.shape[0]
    x2_ref[...] = jnp.sum(x * x, axis=1, keepdims=True)

    @pl.when(i == 0)
    def _():
        sumx_ref[...] = jnp.zeros_like(sumx_ref)
        cnt_ref[...] = jnp.zeros_like(cnt_ref)

    b2 = b_ref[0]
    seg = lax.broadcasted_iota(jnp.int32, (bn, NSEG), 1)
    mask_t = (b2 == seg).astype(jnp.float32)
    sumx_ref[...] += lax.dot_general(
        mask_t, x, (((0,), (0,)), ((), ())),
        preferred_element_type=jnp.float32)
    cnt_ref[...] += lax.dot_general(
        mask_t, jnp.ones((bn, 1), jnp.float32), (((0,), (0,)), ((), ())),
        preferred_element_type=jnp.float32)


def _xstats_call(x, batch, bn):
    n, d = x.shape
    nn = n // bn
    b3 = batch.reshape(nn, bn, 1)
    return pl.pallas_call(
        functools.partial(_xstats_body, nn),
        grid=(nn,),
        in_specs=[
            pl.BlockSpec((bn, d), lambda i: (i, 0)),
            pl.BlockSpec((1, bn, 1), lambda i: (i, 0, 0)),
        ],
        out_specs=[
            pl.BlockSpec((bn, 1), lambda i: (i, 0)),
            pl.BlockSpec((NSEG, d), lambda i: (0, 0)),
            pl.BlockSpec((NSEG, 1), lambda i: (0, 0)),
        ],
        out_shape=[
            jax.ShapeDtypeStruct((n, 1), jnp.float32),
            jax.ShapeDtypeStruct((NSEG, d), jnp.float32),
            jax.ShapeDtypeStruct((NSEG, 1), jnp.float32),
        ],
    )(x, b3)


def _c2_body(cc_ref, c2_ref):
    cc = cc_ref[...]
    c2_ref[...] = jnp.sum(cc * cc, axis=1)[None, :]


def _c2_call(cc, bk):
    k, d = cc.shape
    nk = k // bk
    return pl.pallas_call(
        _c2_body,
        grid=(nk,),
        in_specs=[pl.BlockSpec((bk, d), lambda j: (j, 0))],
        out_specs=pl.BlockSpec((1, bk), lambda j: (0, j)),
        out_shape=jax.ShapeDtypeStruct((1, k), jnp.float32),
    )(cc)


def _argmin_body(nk, bk, x_ref, cc_ref, x2_ref, c2_ref, idx_ref,
                 minval, minidx):
    j = pl.program_id(1)
    mm = lax.dot_general(x_ref[...], cc_ref[...], (((1,), (1,)), ((), ())),
                         preferred_element_type=jnp.float32)
    scores = x2_ref[...] + c2_ref[...] - 2.0 * mm
    bm = jnp.min(scores, axis=1, keepdims=True)
    col = lax.broadcasted_iota(jnp.int32, scores.shape, 1)
    bidx = jnp.min(jnp.where(scores == bm, col, jnp.int32(2 ** 30)),
                   axis=1, keepdims=True) + j * bk
    better = jnp.logical_or(j == 0, bm < minval[...])
    nv = jnp.where(better, bm, minval[...])
    ni = jnp.where(better, bidx, minidx[...])
    minval[...] = nv
    minidx[...] = ni
    idx_ref[0] = ni


def _argmin_call(x, cc, x2, c2, bn, bk):
    n, d = x.shape
    k = cc.shape[0]
    nn, nk = n // bn, k // bk
    return pl.pallas_call(
        functools.partial(_argmin_body, nk, bk),
        grid=(nn, nk),
        in_specs=[
            pl.BlockSpec((bn, d), lambda i, j: (i, 0)),
            pl.BlockSpec((bk, d), lambda i, j: (j, 0)),
            pl.BlockSpec((bn, 1), lambda i, j: (i, 0)),
            pl.BlockSpec((1, bk), lambda i, j: (0, j)),
        ],
        out_specs=pl.BlockSpec((1, bn, 1), lambda i, j: (i, 0, 0)),
        out_shape=jax.ShapeDtypeStruct((nn, bn, 1), jnp.int32),
        scratch_shapes=[
            pltpu.VMEM((bn, 1), jnp.float32),
            pltpu.VMEM((bn, 1), jnp.int32),
        ],
    )(x, cc, x2, c2)


def _sc_gather3(idx, t0, t1, t2):
    n = idx.shape[0]
    d = t0.shape[1]
    nw = 32
    rw = 320
    ch = 80
    assert n == 31 * rw + ch and rw % ch == 0

    mesh = plsc.VectorSubcoreMesh(core_axis_name="c", subcore_axis_name="s",
                                  num_cores=2, num_subcores=16)
    out_t = [jax.ShapeDtypeStruct((n, d), jnp.float32)] * 3

    @functools.partial(
        pl.kernel, out_type=out_t, mesh=mesh,
        scratch_types=[
            pltpu.VMEM((ch,), jnp.int32),
            pltpu.VMEM((ch, d), jnp.float32),
            pltpu.VMEM((ch, d), jnp.float32),
            pltpu.VMEM((ch, d), jnp.float32),
            pltpu.SemaphoreType.DMA,
            pltpu.SemaphoreType.DMA,
            pltpu.SemaphoreType.DMA,
        ],
    )
    def k(idx_hbm, t0_hbm, t1_hbm, t2_hbm, o0_hbm, o1_hbm, o2_hbm,
          idx_v, r0, r1, r2, s0, s1, s2):
        wid = lax.axis_index("s") * 2 + lax.axis_index("c")
        base = wid * rw
        for c in range(rw // ch):
            off = jnp.minimum(base + c * ch, n - ch)
            pltpu.sync_copy(idx_hbm.at[pl.ds(off, ch)], idx_v)
            cp0 = pltpu.make_async_copy(t0_hbm.at[idx_v], r0, s0)
            cp1 = pltpu.make_async_copy(t1_hbm.at[idx_v], r1, s1)
            cp2 = pltpu.make_async_copy(t2_hbm.at[idx_v], r2, s2)
            cp0.start(); cp1.start(); cp2.start()
            cp0.wait(); cp1.wait(); cp2.wait()
            pltpu.sync_copy(r0, o0_hbm.at[pl.ds(off, ch)])
            pltpu.sync_copy(r1, o1_hbm.at[pl.ds(off, ch)])
            pltpu.sync_copy(r2, o2_hbm.at[pl.ds(off, ch)])

    return k(idx, t0, t1, t2)


def _pool_body(nn, causal_ref, counter_ref, b_ref, sumx_ref, cnt_ref,
               w_ref, bias_ref,
               cpre_ref, kpre_ref, ypre_ref, pc_ref, px_ref,
               acc_c, acc_k):
    i = pl.program_id(0)
    b2 = b_ref[0]
    bn = b2.shape[0]
    seg = lax.broadcasted_iota(jnp.int32, (bn, NSEG), 1)
    mask_t = (b2 == seg).astype(jnp.float32)

    @pl.when(i == 0)
    def _():
        acc_c[...] = jnp.zeros_like(acc_c)
        acc_k[...] = jnp.zeros_like(acc_k)

    acc_c[...] += lax.dot_general(mask_t, causal_ref[...],
                                  (((0,), (0,)), ((), ())),
                                  preferred_element_type=jnp.float32)
    acc_k[...] += lax.dot_general(mask_t, counter_ref[...],
                                  (((0,), (0,)), ((), ())),
                                  preferred_element_type=jnp.float32)

    @pl.when(i == nn - 1)
    def _():
        cnt = jnp.maximum(cnt_ref[...], 1.0)
        pooled_x = sumx_ref[...] / cnt
        pooled_c = pooled_x + acc_c[...] / cnt
        pooled_k = acc_k[...] / cnt
        w = w_ref[...]
        bias = bias_ref[...]
        dn = (((1,), (1,)), ((), ()))
        cpre_ref[...] = lax.dot_general(
            pooled_c, w, dn, preferred_element_type=jnp.float32) + bias
        kpre_ref[...] = lax.dot_general(
            pooled_k, w, dn, preferred_element_type=jnp.float32) + bias
        ypre_ref[...] = lax.dot_general(
            pooled_x, w, dn, preferred_element_type=jnp.float32) + bias
        pc_ref[...] = pooled_c
        px_ref[...] = pooled_x


def _pool_call(causal_rows, counter_rows, batch, sumx, cnt, w, bias, bn):
    n, d = causal_rows.shape
    t = w.shape[0]
    nn = n // bn
    b3 = batch.reshape(nn, bn, 1)
    whole = lambda shape: pl.BlockSpec(shape, lambda i: tuple(0 for _ in shape))
    return pl.pallas_call(
        functools.partial(_pool_body, nn),
        grid=(nn,),
        in_specs=[
            pl.BlockSpec((bn, d), lambda i: (i, 0)),
            pl.BlockSpec((bn, d), lambda i: (i, 0)),
            pl.BlockSpec((1, bn, 1), lambda i: (i, 0, 0)),
            whole((NSEG, d)),
            whole((NSEG, 1)),
            whole((t, d)),
            whole((1, t)),
        ],
        out_specs=[
            whole((NSEG, t)),
            whole((NSEG, t)),
            whole((NSEG, t)),
            whole((NSEG, d)),
            whole((NSEG, d)),
        ],
        out_shape=[
            jax.ShapeDtypeStruct((NSEG, t), jnp.float32),
            jax.ShapeDtypeStruct((NSEG, t), jnp.float32),
            jax.ShapeDtypeStruct((NSEG, t), jnp.float32),
            jax.ShapeDtypeStruct((NSEG, d), jnp.float32),
            jax.ShapeDtypeStruct((NSEG, d), jnp.float32),
        ],
        scratch_shapes=[
            pltpu.VMEM((NSEG, d), jnp.float32),
            pltpu.VMEM((NSEG, d), jnp.float32),
        ],
    )(causal_rows, counter_rows, b3, sumx, cnt, w, bias)


def kernel(x, batch, codebook, causal_codebook, counter_codebook, W, b):
    n, d = x.shape
    batch = batch.astype(jnp.int32)
    bn = 1000
    bk = 1024

    x2, sumx, cnt = _xstats_call(x, batch, bn)
    c2 = _c2_call(causal_codebook, bk)
    idx3 = _argmin_call(x, causal_codebook, x2, c2, 2000, bk)
    idx = idx3.reshape(n)

    z_nodes, causal_rows, counter_rows = _sc_gather3(
        idx, codebook, causal_codebook, counter_codebook)

    causal_pre, counter_pre, y_pre, pooled_causal, pooled_x = _pool_call(
        causal_rows, counter_rows, batch, sumx, cnt, W,
        b.reshape(1, -1), bn)

    return (causal_pre, counter_pre, y_pre, z_nodes, pooled_causal, pooled_x)


# speedup vs baseline: 1.2173x; 1.0229x over previous
import functools

import jax
import jax.numpy as jnp
from jax import lax
from jax.experimental import pallas as pl
from jax.experimental.pallas import tpu as pltpu
from jax.experimental.pallas import tpu_sc as plsc

NSEG = 256


def _xstats_body(nn, x_ref, b_ref, x2_ref, sumx_ref, cnt_ref):
    i = pl.program_id(0)
    x = x_ref[...]
    bn = x.shape[0]
    x2_ref[...] = jnp.sum(x * x, axis=1, keepdims=True)

    @pl.when(i == 0)
    def _():
        sumx_ref[...] = jnp.zeros_like(sumx_ref)
        cnt_ref[...] = jnp.zeros_like(cnt_ref)

    b2 = b_ref[0]
    seg = lax.broadcasted_iota(jnp.int32, (bn, NSEG), 1)
    mask_t = (b2 == seg).astype(jnp.float32)
    sumx_ref[...] += lax.dot_general(
        mask_t, x, (((0,), (0,)), ((), ())),
        preferred_element_type=jnp.float32)
    cnt_ref[...] += lax.dot_general(
        mask_t, jnp.ones((bn, 1), jnp.float32), (((0,), (0,)), ((), ())),
        preferred_element_type=jnp.float32)


def _xstats_call(x, batch, bn):
    n, d = x.shape
    nn = n // bn
    b3 = batch.reshape(nn, bn, 1)
    return pl.pallas_call(
        functools.partial(_xstats_body, nn),
        grid=(nn,),
        in_specs=[
            pl.BlockSpec((bn, d), lambda i: (i, 0)),
            pl.BlockSpec((1, bn, 1), lambda i: (i, 0, 0)),
        ],
        out_specs=[
            pl.BlockSpec((bn, 1), lambda i: (i, 0)),
            pl.BlockSpec((NSEG, d), lambda i: (0, 0)),
            pl.BlockSpec((NSEG, 1), lambda i: (0, 0)),
        ],
        out_shape=[
            jax.ShapeDtypeStruct((n, 1), jnp.float32),
            jax.ShapeDtypeStruct((NSEG, d), jnp.float32),
            jax.ShapeDtypeStruct((NSEG, 1), jnp.float32),
        ],
    )(x, b3)


def _c2_body(cc_ref, c2_ref):
    cc = cc_ref[...]
    c2_ref[...] = jnp.sum(cc * cc, axis=1)[None, :]


def _c2_call(cc, bk):
    k, d = cc.shape
    nk = k // bk
    return pl.pallas_call(
        _c2_body,
        grid=(nk,),
        in_specs=[pl.BlockSpec((bk, d), lambda j: (j, 0))],
        out_specs=pl.BlockSpec((1, bk), lambda j: (0, j)),
        out_shape=jax.ShapeDtypeStruct((1, k), jnp.float32),
    )(cc)


def _argmin_body(nk, bk, x_ref, cc_ref, x2_ref, c2_ref, idx_ref,
                 minval, minidx):
    j = pl.program_id(1)
    mm = lax.dot_general(x_ref[...], cc_ref[...], (((1,), (1,)), ((), ())),
                         preferred_element_type=jnp.float32)
    scores = x2_ref[...] + c2_ref[...] - 2.0 * mm
    bm = jnp.min(scores, axis=1, keepdims=True)
    col = lax.broadcasted_iota(jnp.int32, scores.shape, 1)
    bidx = jnp.min(jnp.where(scores == bm, col, jnp.int32(2 ** 30)),
                   axis=1, keepdims=True) + j * bk
    better = jnp.logical_or(j == 0, bm < minval[...])
    nv = jnp.where(better, bm, minval[...])
    ni = jnp.where(better, bidx, minidx[...])
    minval[...] = nv
    minidx[...] = ni
    idx_ref[0] = ni


def _argmin_call(x, cc, x2, c2, bn, bk):
    n, d = x.shape
    k = cc.shape[0]
    nn, nk = n // bn, k // bk
    return pl.pallas_call(
        functools.partial(_argmin_body, nk, bk),
        grid=(nn, nk),
        in_specs=[
            pl.BlockSpec((bn, d), lambda i, j: (i, 0)),
            pl.BlockSpec((bk, d), lambda i, j: (j, 0)),
            pl.BlockSpec((bn, 1), lambda i, j: (i, 0)),
            pl.BlockSpec((1, bk), lambda i, j: (0, j)),
        ],
        out_specs=pl.BlockSpec((1, bn, 1), lambda i, j: (i, 0, 0)),
        out_shape=jax.ShapeDtypeStruct((nn, bn, 1), jnp.int32),
        scratch_shapes=[
            pltpu.VMEM((bn, 1), jnp.float32),
            pltpu.VMEM((bn, 1), jnp.int32),
        ],
    )(x, cc, x2, c2)


def _sc_mesh():
    return plsc.VectorSubcoreMesh(core_axis_name="c", subcore_axis_name="s",
                                  num_cores=2, num_subcores=16)


def _sc_gather1(idx, t0):
    n = idx.shape[0]
    d = t0.shape[1]
    rw = 320
    ch = 80
    assert n == 31 * rw + ch and rw % ch == 0

    @functools.partial(
        pl.kernel, out_type=jax.ShapeDtypeStruct((n, d), jnp.float32),
        mesh=_sc_mesh(),
        scratch_types=[
            pltpu.VMEM((ch,), jnp.int32),
            pltpu.VMEM((ch, d), jnp.float32),
            pltpu.VMEM((ch, d), jnp.float32),
            pltpu.SemaphoreType.DMA,
            pltpu.SemaphoreType.DMA,
            pltpu.SemaphoreType.DMA,
        ],
    )
    def k(idx_hbm, t0_hbm, o0_hbm, idx_v, r0, r1, sg, st0, st1):
        wid = lax.axis_index("s") * 2 + lax.axis_index("c")
        base = wid * rw
        bufs = (r0, r1)
        sts = (st0, st1)
        offs = []
        for c in range(rw // ch):
            off = jnp.minimum(base + c * ch, n - ch)
            offs.append(off)
            b = bufs[c % 2]
            if c >= 2:
                pltpu.make_async_copy(bufs[c % 2], o0_hbm.at[pl.ds(offs[c - 2], ch)],
                                      sts[c % 2]).wait()
            pltpu.sync_copy(idx_hbm.at[pl.ds(off, ch)], idx_v)
            pltpu.make_async_copy(t0_hbm.at[idx_v], b, sg).start()
            pltpu.make_async_copy(t0_hbm.at[idx_v], b, sg).wait()
            pltpu.make_async_copy(b, o0_hbm.at[pl.ds(off, ch)], sts[c % 2]).start()
        for c in (2, 3):
            pltpu.make_async_copy(bufs[c % 2], o0_hbm.at[pl.ds(offs[c], ch)],
                                  sts[c % 2]).wait()

    return k(idx, t0)


def _sc_pbuild(idx, batch, nseg, k):
    n = idx.shape[0]
    nw = 32
    kw = k // nw
    cch = 2000
    nch = n // cch
    assert n % cch == 0 and cch % 16 == 0

    @functools.partial(
        pl.kernel,
        out_type=jax.ShapeDtypeStruct((nw, nseg, kw), jnp.float32),
        mesh=_sc_mesh(),
        compiler_params=pltpu.CompilerParams(needs_layout_passes=False),
        scratch_types=[
            pltpu.VMEM((cch,), jnp.int32),
            pltpu.VMEM((cch,), jnp.int32),
            pltpu.VMEM((nseg, kw), jnp.float32),
        ],
    )
    def kern(idx_hbm, bat_hbm, zero_hbm, pout_hbm, idx_v, bat_v, p_v):
        wid = lax.axis_index("s") * 2 + lax.axis_index("c")
        kbase = wid * kw
        pltpu.sync_copy(zero_hbm, p_v)
        ones = jnp.ones((16,), jnp.float32)
        for c in range(nch):
            pltpu.sync_copy(idx_hbm.at[pl.ds(c * cch, cch)], idx_v)
            pltpu.sync_copy(bat_hbm.at[pl.ds(c * cch, cch)], bat_v)

            def body(g, carry):
                i16 = idx_v[pl.ds(g * 16, 16)]
                b16 = bat_v[pl.ds(g * 16, 16)]
                lk = i16 - kbase
                m = jnp.logical_and(lk >= 0, lk < kw)
                lkc = jnp.minimum(jnp.maximum(lk, 0), kw - 1)
                plsc.addupdate_scatter(p_v, [b16, lkc], ones, mask=m)
                return carry

            lax.fori_loop(0, cch // 16, body, 0)
        pltpu.sync_copy(p_v, pout_hbm.at[wid])

    return kern(idx, batch, jnp.zeros((nseg, kw), jnp.float32))


def _pool_body(nw, p_ref, causal_ref, counter_ref, sumx_ref, cnt_ref,
               w_ref, bias_ref,
               cpre_ref, kpre_ref, ypre_ref, pc_ref, px_ref,
               acc_c, acc_k):
    i = pl.program_id(0)
    pblk = p_ref[0]

    @pl.when(i == 0)
    def _():
        acc_c[...] = jnp.zeros_like(acc_c)
        acc_k[...] = jnp.zeros_like(acc_k)

    dn = (((1,), (0,)), ((), ()))
    acc_c[...] += lax.dot_general(pblk, causal_ref[...], dn,
                                  preferred_element_type=jnp.float32)
    acc_k[...] += lax.dot_general(pblk, counter_ref[...], dn,
                                  preferred_element_type=jnp.float32)

    @pl.when(i == nw - 1)
    def _():
        cnt = jnp.maximum(cnt_ref[...], 1.0)
        pooled_x = sumx_ref[...] / cnt
        pooled_c = pooled_x + acc_c[...] / cnt
        pooled_k = acc_k[...] / cnt
        w = w_ref[...]
        bias = bias_ref[...]
        dh = (((1,), (1,)), ((), ()))
        cpre_ref[...] = lax.dot_general(
            pooled_c, w, dh, preferred_element_type=jnp.float32) + bias
        kpre_ref[...] = lax.dot_general(
            pooled_k, w, dh, preferred_element_type=jnp.float32) + bias
        ypre_ref[...] = lax.dot_general(
            pooled_x, w, dh, preferred_element_type=jnp.float32) + bias
        pc_ref[...] = pooled_c
        px_ref[...] = pooled_x


def _pool_call(pmat, causal, counter, sumx, cnt, w, bias):
    nw, nseg, kw = pmat.shape
    d = causal.shape[1]
    t = w.shape[0]
    whole = lambda shape: pl.BlockSpec(shape, lambda i: tuple(0 for _ in shape))
    return pl.pallas_call(
        functools.partial(_pool_body, nw),
        grid=(nw,),
        in_specs=[
            pl.BlockSpec((1, nseg, kw), lambda i: (i, 0, 0)),
            pl.BlockSpec((kw, d), lambda i: (i, 0)),
            pl.BlockSpec((kw, d), lambda i: (i, 0)),
            whole((NSEG, d)),
            whole((NSEG, 1)),
            whole((t, d)),
            whole((1, t)),
        ],
        out_specs=[
            whole((NSEG, t)),
            whole((NSEG, t)),
            whole((NSEG, t)),
            whole((NSEG, d)),
            whole((NSEG, d)),
        ],
        out_shape=[
            jax.ShapeDtypeStruct((NSEG, t), jnp.float32),
            jax.ShapeDtypeStruct((NSEG, t), jnp.float32),
            jax.ShapeDtypeStruct((NSEG, t), jnp.float32),
            jax.ShapeDtypeStruct((NSEG, d), jnp.float32),
            jax.ShapeDtypeStruct((NSEG, d), jnp.float32),
        ],
        scratch_shapes=[
            pltpu.VMEM((NSEG, d), jnp.float32),
            pltpu.VMEM((NSEG, d), jnp.float32),
        ],
    )(pmat, causal, counter, sumx, cnt, w, bias)


def kernel(x, batch, codebook, causal_codebook, counter_codebook, W, b):
    n, d = x.shape
    batch = batch.astype(jnp.int32)
    bn = 1000
    bk = 1024

    x2, sumx, cnt = _xstats_call(x, batch, bn)
    c2 = _c2_call(causal_codebook, bk)
    idx3 = _argmin_call(x, causal_codebook, x2, c2, 2000, bk)
    idx = idx3.reshape(n)

    pmat = _sc_pbuild(idx, batch, NSEG, causal_codebook.shape[0])
    z_nodes = _sc_gather1(idx, codebook)

    causal_pre, counter_pre, y_pre, pooled_causal, pooled_x = _pool_call(
        pmat, causal_codebook, counter_codebook, sumx, cnt, W,
        b.reshape(1, -1))

    return (causal_pre, counter_pre, y_pre, z_nodes, pooled_causal, pooled_x)


# speedup vs baseline: 1.2733x; 1.0460x over previous
import functools

import jax
import jax.numpy as jnp
from jax import lax
from jax.experimental import pallas as pl
from jax.experimental.pallas import tpu as pltpu
from jax.experimental.pallas import tpu_sc as plsc

NSEG = 256


def _xstats_body(nn, x_ref, b_ref, x2_ref, sumx_ref, cnt_ref):
    i = pl.program_id(0)
    x = x_ref[...]
    bn = x.shape[0]
    x2_ref[...] = jnp.sum(x * x, axis=1, keepdims=True)

    @pl.when(i == 0)
    def _():
        sumx_ref[...] = jnp.zeros_like(sumx_ref)
        cnt_ref[...] = jnp.zeros_like(cnt_ref)

    b2 = b_ref[0]
    seg = lax.broadcasted_iota(jnp.int32, (bn, NSEG), 1)
    mask_t = (b2 == seg).astype(jnp.float32)
    sumx_ref[...] += lax.dot_general(
        mask_t, x, (((0,), (0,)), ((), ())),
        preferred_element_type=jnp.float32)
    cnt_ref[...] += lax.dot_general(
        mask_t, jnp.ones((bn, 1), jnp.float32), (((0,), (0,)), ((), ())),
        preferred_element_type=jnp.float32)


def _xstats_call(x, batch, bn):
    n, d = x.shape
    nn = n // bn
    b3 = batch.reshape(nn, bn, 1)
    return pl.pallas_call(
        functools.partial(_xstats_body, nn),
        grid=(nn,),
        in_specs=[
            pl.BlockSpec((bn, d), lambda i: (i, 0)),
            pl.BlockSpec((1, bn, 1), lambda i: (i, 0, 0)),
        ],
        out_specs=[
            pl.BlockSpec((bn, 1), lambda i: (i, 0)),
            pl.BlockSpec((NSEG, d), lambda i: (0, 0)),
            pl.BlockSpec((NSEG, 1), lambda i: (0, 0)),
        ],
        out_shape=[
            jax.ShapeDtypeStruct((n, 1), jnp.float32),
            jax.ShapeDtypeStruct((NSEG, d), jnp.float32),
            jax.ShapeDtypeStruct((NSEG, 1), jnp.float32),
        ],
    )(x, b3)


def _c2_body(cc_ref, c2_ref):
    cc = cc_ref[...]
    c2_ref[...] = jnp.sum(cc * cc, axis=1)[None, :]


def _c2_call(cc, bk):
    k, d = cc.shape
    nk = k // bk
    return pl.pallas_call(
        _c2_body,
        grid=(nk,),
        in_specs=[pl.BlockSpec((bk, d), lambda j: (j, 0))],
        out_specs=pl.BlockSpec((1, bk), lambda j: (0, j)),
        out_shape=jax.ShapeDtypeStruct((1, k), jnp.float32),
    )(cc)


def _argmin_body(nk, bk, x_ref, cc_ref, x2_ref, c2_ref, idx_ref,
                 minval, minidx):
    j = pl.program_id(1)
    mm = lax.dot_general(x_ref[...], cc_ref[...], (((1,), (1,)), ((), ())),
                         preferred_element_type=jnp.float32)
    scores = x2_ref[...] + c2_ref[...] - 2.0 * mm
    bm = jnp.min(scores, axis=1, keepdims=True)
    col = lax.broadcasted_iota(jnp.int32, scores.shape, 1)
    bidx = jnp.min(jnp.where(scores == bm, col, jnp.int32(2 ** 30)),
                   axis=1, keepdims=True) + j * bk
    better = jnp.logical_or(j == 0, bm < minval[...])
    nv = jnp.where(better, bm, minval[...])
    ni = jnp.where(better, bidx, minidx[...])
    minval[...] = nv
    minidx[...] = ni
    idx_ref[0] = ni


def _argmin_call(x, cc, x2, c2, bn, bk):
    n, d = x.shape
    k = cc.shape[0]
    nn, nk = n // bn, k // bk
    return pl.pallas_call(
        functools.partial(_argmin_body, nk, bk),
        grid=(nn, nk),
        in_specs=[
            pl.BlockSpec((bn, d), lambda i, j: (i, 0)),
            pl.BlockSpec((bk, d), lambda i, j: (j, 0)),
            pl.BlockSpec((bn, 1), lambda i, j: (i, 0)),
            pl.BlockSpec((1, bk), lambda i, j: (0, j)),
        ],
        out_specs=pl.BlockSpec((1, bn, 1), lambda i, j: (i, 0, 0)),
        out_shape=jax.ShapeDtypeStruct((nn, bn, 1), jnp.int32),
        scratch_shapes=[
            pltpu.VMEM((bn, 1), jnp.float32),
            pltpu.VMEM((bn, 1), jnp.int32),
        ],
    )(x, cc, x2, c2)


def _sc_mesh():
    return plsc.VectorSubcoreMesh(core_axis_name="c", subcore_axis_name="s",
                                  num_cores=2, num_subcores=16)


def _sc_gather1(idx, t0):
    n = idx.shape[0]
    d = t0.shape[1]
    rw = 320
    ch = 80
    assert n == 31 * rw + ch and rw % ch == 0

    @functools.partial(
        pl.kernel, out_type=jax.ShapeDtypeStruct((n, d), jnp.float32),
        mesh=_sc_mesh(),
        scratch_types=[
            pltpu.VMEM((rw,), jnp.int32),
            pltpu.VMEM((ch, d), jnp.float32),
            pltpu.VMEM((ch, d), jnp.float32),
            pltpu.SemaphoreType.DMA,
            pltpu.SemaphoreType.DMA,
            pltpu.SemaphoreType.DMA,
            pltpu.SemaphoreType.DMA,
        ],
    )
    def k(idx_hbm, t0_hbm, o0_hbm, idx_v, r0, r1, sg0, sg1, st0, st1):
        wid = lax.axis_index("s") * 2 + lax.axis_index("c")
        base = wid * rw
        bufs = (r0, r1)
        sgs = (sg0, sg1)
        sts = (st0, st1)
        nch = rw // ch
        offs = [jnp.minimum(base + c * ch, n - ch) for c in range(nch)]
        pltpu.sync_copy(idx_hbm.at[pl.ds(jnp.minimum(base, n - rw), rw)], idx_v)
        gathers = []
        stores = []
        for c in range(nch):
            p = c % 2
            if c >= 2:
                stores[c - 2].wait()
            iv = idx_v.at[pl.ds(
                jnp.minimum(base + c * ch, n - ch) - jnp.minimum(base, n - rw),
                ch)]
            g = pltpu.make_async_copy(t0_hbm.at[iv], bufs[p], sgs[p])
            g.start()
            gathers.append(g)
            if c >= 1:
                gathers[c - 1].wait()
                s = pltpu.make_async_copy(bufs[(c - 1) % 2],
                                          o0_hbm.at[pl.ds(offs[c - 1], ch)],
                                          sts[(c - 1) % 2])
                s.start()
                stores.append(s)
        gathers[nch - 1].wait()
        s = pltpu.make_async_copy(bufs[(nch - 1) % 2],
                                  o0_hbm.at[pl.ds(offs[nch - 1], ch)],
                                  sts[(nch - 1) % 2])
        s.start()
        stores.append(s)
        stores[nch - 2].wait()
        stores[nch - 1].wait()

    return k(idx, t0)


def _sc_pbuild(idx, batch, nseg, k):
    n = idx.shape[0]
    nw = 32
    kw = k // nw
    cch = n
    nch = 1
    assert n % cch == 0 and cch % 16 == 0

    @functools.partial(
        pl.kernel,
        out_type=jax.ShapeDtypeStruct((nw, nseg, kw), jnp.float32),
        mesh=_sc_mesh(),
        compiler_params=pltpu.CompilerParams(needs_layout_passes=False),
        scratch_types=[
            pltpu.VMEM((cch,), jnp.int32),
            pltpu.VMEM((cch,), jnp.int32),
            pltpu.VMEM((nseg, kw), jnp.float32),
            pltpu.SemaphoreType.DMA,
            pltpu.SemaphoreType.DMA,
        ],
    )
    def kern(idx_hbm, bat_hbm, zero_hbm, pout_hbm, idx_v, bat_v, p_v, si, sb):
        wid = lax.axis_index("s") * 2 + lax.axis_index("c")
        kbase = wid * kw
        pltpu.sync_copy(zero_hbm, p_v)
        ones = jnp.ones((16,), jnp.float32)
        for c in range(nch):
            ci = pltpu.make_async_copy(idx_hbm.at[pl.ds(c * cch, cch)],
                                       idx_v, si)
            cb = pltpu.make_async_copy(bat_hbm.at[pl.ds(c * cch, cch)],
                                       bat_v, sb)
            ci.start(); cb.start(); ci.wait(); cb.wait()

            def body(g, carry):
                i16 = idx_v[pl.ds(g * 16, 16)]
                b16 = bat_v[pl.ds(g * 16, 16)]
                lk = i16 - kbase
                m = jnp.logical_and(lk >= 0, lk < kw)
                lkc = jnp.minimum(jnp.maximum(lk, 0), kw - 1)
                plsc.addupdate_scatter(p_v, [b16, lkc], ones, mask=m)
                return carry

            lax.fori_loop(0, cch // 16, body, 0)
        pltpu.sync_copy(p_v, pout_hbm.at[wid])

    return kern(idx, batch, jnp.zeros((nseg, kw), jnp.float32))


def _pool_body(nsteps, sub, kw, p_ref, causal_ref, counter_ref,
               sumx_ref, cnt_ref, w_ref, bias_ref,
               cpre_ref, kpre_ref, ypre_ref, pc_ref, px_ref,
               acc_c, acc_k):
    i = pl.program_id(0)

    @pl.when(i == 0)
    def _():
        acc_c[...] = jnp.zeros_like(acc_c)
        acc_k[...] = jnp.zeros_like(acc_k)

    dn = (((1,), (0,)), ((), ()))
    cb = causal_ref[...]
    kb = counter_ref[...]
    ac = acc_c[...]
    ak = acc_k[...]
    for s in range(sub):
        pblk = p_ref[s]
        ac = ac + lax.dot_general(pblk, cb[s * kw:(s + 1) * kw], dn,
                                  preferred_element_type=jnp.float32)
        ak = ak + lax.dot_general(pblk, kb[s * kw:(s + 1) * kw], dn,
                                  preferred_element_type=jnp.float32)
    acc_c[...] = ac
    acc_k[...] = ak

    @pl.when(i == nsteps - 1)
    def _():
        cnt = jnp.maximum(cnt_ref[...], 1.0)
        pooled_x = sumx_ref[...] / cnt
        pooled_c = pooled_x + acc_c[...] / cnt
        pooled_k = acc_k[...] / cnt
        w = w_ref[...]
        bias = bias_ref[...]
        dh = (((1,), (1,)), ((), ()))
        cpre_ref[...] = lax.dot_general(
            pooled_c, w, dh, preferred_element_type=jnp.float32) + bias
        kpre_ref[...] = lax.dot_general(
            pooled_k, w, dh, preferred_element_type=jnp.float32) + bias
        ypre_ref[...] = lax.dot_general(
            pooled_x, w, dh, preferred_element_type=jnp.float32) + bias
        pc_ref[...] = pooled_c
        px_ref[...] = pooled_x


def _pool_call(pmat, causal, counter, sumx, cnt, w, bias):
    nw, nseg, kw = pmat.shape
    d = causal.shape[1]
    t = w.shape[0]
    sub = 4
    nsteps = nw // sub
    whole = lambda shape: pl.BlockSpec(shape, lambda i: tuple(0 for _ in shape))
    return pl.pallas_call(
        functools.partial(_pool_body, nsteps, sub, kw),
        grid=(nsteps,),
        in_specs=[
            pl.BlockSpec((sub, nseg, kw), lambda i: (i, 0, 0)),
            pl.BlockSpec((sub * kw, d), lambda i: (i, 0)),
            pl.BlockSpec((sub * kw, d), lambda i: (i, 0)),
            whole((NSEG, d)),
            whole((NSEG, 1)),
            whole((t, d)),
            whole((1, t)),
        ],
        out_specs=[
            whole((NSEG, t)),
            whole((NSEG, t)),
            whole((NSEG, t)),
            whole((NSEG, d)),
            whole((NSEG, d)),
        ],
        out_shape=[
            jax.ShapeDtypeStruct((NSEG, t), jnp.float32),
            jax.ShapeDtypeStruct((NSEG, t), jnp.float32),
            jax.ShapeDtypeStruct((NSEG, t), jnp.float32),
            jax.ShapeDtypeStruct((NSEG, d), jnp.float32),
            jax.ShapeDtypeStruct((NSEG, d), jnp.float32),
        ],
        scratch_shapes=[
            pltpu.VMEM((NSEG, d), jnp.float32),
            pltpu.VMEM((NSEG, d), jnp.float32),
        ],
    )(pmat, causal, counter, sumx, cnt, w, bias)


def kernel(x, batch, codebook, causal_codebook, counter_codebook, W, b):
    n, d = x.shape
    batch = batch.astype(jnp.int32)
    bn = 1000
    bk = 1024

    x2, sumx, cnt = _xstats_call(x, batch, bn)
    c2 = _c2_call(causal_codebook, bk)
    idx3 = _argmin_call(x, causal_codebook, x2, c2, 2000, bk)
    idx = idx3.reshape(n)

    pmat = _sc_pbuild(idx, batch, NSEG, causal_codebook.shape[0])
    z_nodes = _sc_gather1(idx, codebook)

    causal_pre, counter_pre, y_pre, pooled_causal, pooled_x = _pool_call(
        pmat, causal_codebook, counter_codebook, sumx, cnt, W,
        b.reshape(1, -1))

    return (causal_pre, counter_pre, y_pre, z_nodes, pooled_causal, pooled_x)


# speedup vs baseline: 1.3005x; 1.0214x over previous
import functools

import jax
import jax.numpy as jnp
from jax import lax
from jax.experimental import pallas as pl
from jax.experimental.pallas import tpu as pltpu
from jax.experimental.pallas import tpu_sc as plsc

NSEG = 256


def _xstats_body(nn, x_ref, b_ref, cc_ref, x2_ref, sumx_ref, cnt_ref, c2_ref):
    i = pl.program_id(0)
    x = x_ref[...]
    bn = x.shape[0]
    x2_ref[...] = jnp.sum(x * x, axis=1, keepdims=True)
    cc = cc_ref[...]
    c2_ref[...] = jnp.sum(cc * cc, axis=1)[None, :]

    @pl.when(i == 0)
    def _():
        sumx_ref[...] = jnp.zeros_like(sumx_ref)
        cnt_ref[...] = jnp.zeros_like(cnt_ref)

    b2 = b_ref[0]
    seg = lax.broadcasted_iota(jnp.int32, (bn, NSEG), 1)
    mask_t = (b2 == seg).astype(jnp.float32)
    sumx_ref[...] += lax.dot_general(
        mask_t, x, (((0,), (0,)), ((), ())),
        preferred_element_type=jnp.float32)
    cnt_ref[...] += lax.dot_general(
        mask_t, jnp.ones((bn, 1), jnp.float32), (((0,), (0,)), ((), ())),
        preferred_element_type=jnp.float32)


def _xstats_call(x, batch, cc, bn, bk):
    n, d = x.shape
    k = cc.shape[0]
    nn = n // bn
    nk = k // bk
    b3 = batch.reshape(nn, bn, 1)
    return pl.pallas_call(
        functools.partial(_xstats_body, nn),
        grid=(nn,),
        in_specs=[
            pl.BlockSpec((bn, d), lambda i: (i, 0)),
            pl.BlockSpec((1, bn, 1), lambda i: (i, 0, 0)),
            pl.BlockSpec((bk, d), lambda i: (jnp.minimum(i, nk - 1), 0)),
        ],
        out_specs=[
            pl.BlockSpec((bn, 1), lambda i: (i, 0)),
            pl.BlockSpec((NSEG, d), lambda i: (0, 0)),
            pl.BlockSpec((NSEG, 1), lambda i: (0, 0)),
            pl.BlockSpec((1, bk), lambda i: (0, jnp.minimum(i, nk - 1))),
        ],
        out_shape=[
            jax.ShapeDtypeStruct((n, 1), jnp.float32),
            jax.ShapeDtypeStruct((NSEG, d), jnp.float32),
            jax.ShapeDtypeStruct((NSEG, 1), jnp.float32),
            jax.ShapeDtypeStruct((1, k), jnp.float32),
        ],
    )(x, b3, cc)


def _argmin_body(nk, bk, x_ref, cc_ref, x2_ref, c2_ref, idx_ref,
                 minval, minidx):
    j = pl.program_id(1)
    mm = lax.dot_general(x_ref[...], cc_ref[...], (((1,), (1,)), ((), ())),
                         preferred_element_type=jnp.float32)
    scores = x2_ref[...] + c2_ref[...] - 2.0 * mm
    bm = jnp.min(scores, axis=1, keepdims=True)
    col = lax.broadcasted_iota(jnp.int32, scores.shape, 1)
    bidx = jnp.min(jnp.where(scores == bm, col, jnp.int32(2 ** 30)),
                   axis=1, keepdims=True) + j * bk
    better = jnp.logical_or(j == 0, bm < minval[...])
    nv = jnp.where(better, bm, minval[...])
    ni = jnp.where(better, bidx, minidx[...])
    minval[...] = nv
    minidx[...] = ni
    idx_ref[0] = ni


def _argmin_call(x, cc, x2, c2, bn, bk):
    n, d = x.shape
    k = cc.shape[0]
    nn, nk = n // bn, k // bk
    return pl.pallas_call(
        functools.partial(_argmin_body, nk, bk),
        grid=(nn, nk),
        in_specs=[
            pl.BlockSpec((bn, d), lambda i, j: (i, 0)),
            pl.BlockSpec((bk, d), lambda i, j: (j, 0)),
            pl.BlockSpec((bn, 1), lambda i, j: (i, 0)),
            pl.BlockSpec((1, bk), lambda i, j: (0, j)),
        ],
        out_specs=pl.BlockSpec((1, bn, 1), lambda i, j: (i, 0, 0)),
        out_shape=jax.ShapeDtypeStruct((nn, bn, 1), jnp.int32),
        scratch_shapes=[
            pltpu.VMEM((bn, 1), jnp.float32),
            pltpu.VMEM((bn, 1), jnp.int32),
        ],
    )(x, cc, x2, c2)


def _sc_mesh():
    return plsc.VectorSubcoreMesh(core_axis_name="c", subcore_axis_name="s",
                                  num_cores=2, num_subcores=16)


def _sc_gather1(idx, t0):
    n = idx.shape[0]
    d = t0.shape[1]
    rw = 320
    ch = 80
    assert n == 31 * rw + ch and rw % ch == 0

    @functools.partial(
        pl.kernel, out_type=jax.ShapeDtypeStruct((n, d), jnp.float32),
        mesh=_sc_mesh(),
        scratch_types=[
            pltpu.VMEM((rw,), jnp.int32),
            pltpu.VMEM((ch, d), jnp.float32),
            pltpu.VMEM((ch, d), jnp.float32),
            pltpu.SemaphoreType.DMA,
            pltpu.SemaphoreType.DMA,
            pltpu.SemaphoreType.DMA,
            pltpu.SemaphoreType.DMA,
        ],
    )
    def k(idx_hbm, t0_hbm, o0_hbm, idx_v, r0, r1, sg0, sg1, st0, st1):
        wid = lax.axis_index("s") * 2 + lax.axis_index("c")
        base = wid * rw
        bufs = (r0, r1)
        sgs = (sg0, sg1)
        sts = (st0, st1)
        nch = rw // ch
        offs = [jnp.minimum(base + c * ch, n - ch) for c in range(nch)]
        pltpu.sync_copy(idx_hbm.at[pl.ds(jnp.minimum(base, n - rw), rw)], idx_v)
        gathers = []
        stores = []
        for c in range(nch):
            p = c % 2
            if c >= 2:
                stores[c - 2].wait()
            iv = idx_v.at[pl.ds(
                jnp.minimum(base + c * ch, n - ch) - jnp.minimum(base, n - rw),
                ch)]
            g = pltpu.make_async_copy(t0_hbm.at[iv], bufs[p], sgs[p])
            g.start()
            gathers.append(g)
            if c >= 1:
                gathers[c - 1].wait()
                s = pltpu.make_async_copy(bufs[(c - 1) % 2],
                                          o0_hbm.at[pl.ds(offs[c - 1], ch)],
                                          sts[(c - 1) % 2])
                s.start()
                stores.append(s)
        gathers[nch - 1].wait()
        s = pltpu.make_async_copy(bufs[(nch - 1) % 2],
                                  o0_hbm.at[pl.ds(offs[nch - 1], ch)],
                                  sts[(nch - 1) % 2])
        s.start()
        stores.append(s)
        stores[nch - 2].wait()
        stores[nch - 1].wait()

    return k(idx, t0)


def _sc_pbuild(idx, batch, nseg, k):
    n = idx.shape[0]
    nw = 32
    kw = k // nw
    cch = n
    nch = 1
    assert n % cch == 0 and cch % 16 == 0

    @functools.partial(
        pl.kernel,
        out_type=jax.ShapeDtypeStruct((nw, nseg, kw), jnp.float32),
        mesh=_sc_mesh(),
        compiler_params=pltpu.CompilerParams(needs_layout_passes=False),
        scratch_types=[
            pltpu.VMEM((cch,), jnp.int32),
            pltpu.VMEM((cch,), jnp.int32),
            pltpu.VMEM((nseg, kw), jnp.float32),
            pltpu.SemaphoreType.DMA,
            pltpu.SemaphoreType.DMA,
        ],
    )
    def kern(idx_hbm, bat_hbm, zero_hbm, pout_hbm, idx_v, bat_v, p_v, si, sb):
        wid = lax.axis_index("s") * 2 + lax.axis_index("c")
        kbase = wid * kw
        pltpu.sync_copy(zero_hbm, p_v)
        ones = jnp.ones((16,), jnp.float32)
        for c in range(nch):
            ci = pltpu.make_async_copy(idx_hbm.at[pl.ds(c * cch, cch)],
                                       idx_v, si)
            cb = pltpu.make_async_copy(bat_hbm.at[pl.ds(c * cch, cch)],
                                       bat_v, sb)
            ci.start(); cb.start(); ci.wait(); cb.wait()
            unroll = 5
            assert cch % (16 * unroll) == 0

            def body(g, carry):
                for u in range(unroll):
                    o = g * (16 * unroll) + u * 16
                    i16 = idx_v[pl.ds(o, 16)]
                    b16 = bat_v[pl.ds(o, 16)]
                    lk = i16 - kbase
                    m = jnp.logical_and(lk >= 0, lk < kw)
                    lkc = jnp.minimum(jnp.maximum(lk, 0), kw - 1)
                    plsc.addupdate_scatter(p_v, [b16, lkc], ones, mask=m)
                return carry

            lax.fori_loop(0, cch // (16 * unroll), body, 0)
        pltpu.sync_copy(p_v, pout_hbm.at[wid])

    return kern(idx, batch, jnp.zeros((nseg, kw), jnp.float32))


def _pool_body(nsteps, sub, kw, p_ref, causal_ref, counter_ref,
               sumx_ref, cnt_ref, w_ref, bias_ref,
               cpre_ref, kpre_ref, ypre_ref, pc_ref, px_ref,
               acc_c, acc_k):
    i = pl.program_id(0)

    @pl.when(i == 0)
    def _():
        acc_c[...] = jnp.zeros_like(acc_c)
        acc_k[...] = jnp.zeros_like(acc_k)

    dn = (((1,), (0,)), ((), ()))
    cb = causal_ref[...]
    kb = counter_ref[...]
    ac = acc_c[...]
    ak = acc_k[...]
    for s in range(sub):
        pblk = p_ref[s]
        ac = ac + lax.dot_general(pblk, cb[s * kw:(s + 1) * kw], dn,
                                  preferred_element_type=jnp.float32)
        ak = ak + lax.dot_general(pblk, kb[s * kw:(s + 1) * kw], dn,
                                  preferred_element_type=jnp.float32)
    acc_c[...] = ac
    acc_k[...] = ak

    @pl.when(i == nsteps - 1)
    def _():
        cnt = jnp.maximum(cnt_ref[...], 1.0)
        pooled_x = sumx_ref[...] / cnt
        pooled_c = pooled_x + acc_c[...] / cnt
        pooled_k = acc_k[...] / cnt
        w = w_ref[...]
        bias = bias_ref[...]
        dh = (((1,), (1,)), ((), ()))
        cpre_ref[...] = lax.dot_general(
            pooled_c, w, dh, preferred_element_type=jnp.float32) + bias
        kpre_ref[...] = lax.dot_general(
            pooled_k, w, dh, preferred_element_type=jnp.float32) + bias
        ypre_ref[...] = lax.dot_general(
            pooled_x, w, dh, preferred_element_type=jnp.float32) + bias
        pc_ref[...] = pooled_c
        px_ref[...] = pooled_x


def _pool_call(pmat, causal, counter, sumx, cnt, w, bias):
    nw, nseg, kw = pmat.shape
    d = causal.shape[1]
    t = w.shape[0]
    sub = 4
    nsteps = nw // sub
    whole = lambda shape: pl.BlockSpec(shape, lambda i: tuple(0 for _ in shape))
    return pl.pallas_call(
        functools.partial(_pool_body, nsteps, sub, kw),
        grid=(nsteps,),
        in_specs=[
            pl.BlockSpec((sub, nseg, kw), lambda i: (i, 0, 0)),
            pl.BlockSpec((sub * kw, d), lambda i: (i, 0)),
            pl.BlockSpec((sub * kw, d), lambda i: (i, 0)),
            whole((NSEG, d)),
            whole((NSEG, 1)),
            whole((t, d)),
            whole((1, t)),
        ],
        out_specs=[
            whole((NSEG, t)),
            whole((NSEG, t)),
            whole((NSEG, t)),
            whole((NSEG, d)),
            whole((NSEG, d)),
        ],
        out_shape=[
            jax.ShapeDtypeStruct((NSEG, t), jnp.float32),
            jax.ShapeDtypeStruct((NSEG, t), jnp.float32),
            jax.ShapeDtypeStruct((NSEG, t), jnp.float32),
            jax.ShapeDtypeStruct((NSEG, d), jnp.float32),
            jax.ShapeDtypeStruct((NSEG, d), jnp.float32),
        ],
        scratch_shapes=[
            pltpu.VMEM((NSEG, d), jnp.float32),
            pltpu.VMEM((NSEG, d), jnp.float32),
        ],
    )(pmat, causal, counter, sumx, cnt, w, bias)


def kernel(x, batch, codebook, causal_codebook, counter_codebook, W, b):
    n, d = x.shape
    batch = batch.astype(jnp.int32)
    bn = 1000
    bk = 1024

    x2, sumx, cnt, c2 = _xstats_call(x, batch, causal_codebook, bn, bk)
    idx3 = _argmin_call(x, causal_codebook, x2, c2, 2000, bk)
    idx = idx3.reshape(n)

    pmat = _sc_pbuild(idx, batch, NSEG, causal_codebook.shape[0])
    z_nodes = _sc_gather1(idx, codebook)

    causal_pre, counter_pre, y_pre, pooled_causal, pooled_x = _pool_call(
        pmat, causal_codebook, counter_codebook, sumx, cnt, W,
        b.reshape(1, -1))

    return (causal_pre, counter_pre, y_pre, z_nodes, pooled_causal, pooled_x)
